# baseline (speedup 1.0000x reference)
import functools

import jax
import jax.numpy as jnp
from jax import lax
from jax.experimental import pallas as pl
from jax.experimental.pallas import tpu as pltpu
from jax.experimental.pallas import tpu_sc as plsc

_NUM_BINS = 1000
_D = 64
_BATCH = 16384
_HIST = 200
_TOTAL = _BATCH * _HIST

_NC = 2
_NS = 16
_NW = _NC * _NS
_BW = _BATCH // _NW
_CB = 4
_R = _CB * _HIST
_G = 80
_NJ = _R // _G
_NB = _BW // _CB


def _body(iv_hbm, tab_hbm, out_hbm, iv, idx, rows, gsem0, gsem1, osem0, osem1):
    wid = lax.axis_index("s") * _NC + lax.axis_index("c")
    batch0 = wid * _BW
    gsems = (gsem0, gsem1)
    osems = (osem0, osem1)

    def stage(g, b):
        start = (batch0 + g * _CB) * _HIST
        pltpu.sync_copy(iv_hbm.at[pl.ds(start, _R)], iv.at[b])
        iv_b = iv.at[b]
        for j in range(_NJ):
            idx_bj = idx.at[b].at[j]
            for i in range(_G // 16):
                v = iv_b[pl.ds(j * _G + i * 16, 16)]
                bins = jnp.minimum(
                    lax.shift_right_logical(v * 37450, 18), _NUM_BINS - 1
                )
                idx_bj[pl.ds(i * 16, 16)] = bins
        for j in range(_NJ):
            pltpu.async_copy(
                tab_hbm.at[idx.at[b].at[j]],
                rows.at[b].at[pl.ds(j * _G, _G)],
                gsems[b],
            )

    def drain_gathers(b):
        for j in range(_NJ):
            pltpu.make_async_copy(
                tab_hbm.at[idx.at[b].at[j]],
                rows.at[b].at[pl.ds(j * _G, _G)],
                gsems[b],
            ).wait()

    def fire_out(g, b):
        for e in range(_CB):
            pltpu.async_copy(
                rows.at[b].at[pl.ds(e * _HIST, _HIST)],
                out_hbm.at[batch0 + g * _CB + e],
                osems[b],
            )

    def drain_out(g, b):
        for e in range(_CB):
            pltpu.make_async_copy(
                rows.at[b].at[pl.ds(e * _HIST, _HIST)],
                out_hbm.at[batch0 + g * _CB + e],
                osems[b],
            ).wait()

    stage(0, 0)
    stage(1, 1)

    def loop_body(i, carry):
        g = i * 2
        drain_gathers(0)
        fire_out(g, 0)
        drain_gathers(1)
        fire_out(g + 1, 1)

        @pl.when(g + 2 < _NB)
        def _():
            drain_out(g, 0)
            stage(g + 2, 0)
            drain_out(g + 1, 1)
            stage(g + 3, 1)

        return carry

    lax.fori_loop(0, _NB // 2, loop_body, 0)
    drain_out(_NB - 2, 0)
    drain_out(_NB - 1, 1)


_sc_lookup = functools.partial(
    pl.kernel,
    out_type=jax.ShapeDtypeStruct((_BATCH, _HIST, _D), jnp.float32),
    mesh=plsc.VectorSubcoreMesh(core_axis_name="c", subcore_axis_name="s"),
    compiler_params=pltpu.CompilerParams(use_tc_tiling_on_sc=False),
    scratch_types=[
        pltpu.VMEM((2, _R), jnp.int32),
        pltpu.VMEM((2, _NJ, _G), jnp.int32),
        pltpu.VMEM((2, _R, _D), jnp.float32),
        pltpu.SemaphoreType.DMA,
        pltpu.SemaphoreType.DMA,
        pltpu.SemaphoreType.DMA,
        pltpu.SemaphoreType.DMA,
    ],
)(_body)


@jax.jit
def kernel(intervals, embed_weight):
    return _sc_lookup(intervals.reshape(_TOTAL), embed_weight)


# speedup vs baseline: 1.0013x; 1.0013x over previous
import functools

import jax
import jax.experimental.layout
import jax.numpy as jnp
from jax import lax
from jax.experimental import pallas as pl
from jax.experimental.pallas import tpu as pltpu
from jax.experimental.pallas import tpu_sc as plsc

_NUM_BINS = 1000
_D = 64
_BATCH = 16384
_HIST = 200
_TOTAL = _BATCH * _HIST

_NC = 2
_NS = 16
_NW = _NC * _NS
_BW = _BATCH // _NW
_CB = 4
_R = _CB * _HIST
_G = 80
_NJ = _R // _G
_NB = _BW // _CB


def _body(iv_hbm, tab_hbm, out_hbm, iv, idx, rows, gsem0, gsem1, osem0, osem1):
    wid = lax.axis_index("s") * _NC + lax.axis_index("c")
    batch0 = wid * _BW
    gsems = (gsem0, gsem1)
    osems = (osem0, osem1)

    def stage(g, b):
        start = (batch0 + g * _CB) * _HIST
        pltpu.sync_copy(iv_hbm.at[pl.ds(start, _R)], iv.at[b])
        iv_b = iv.at[b]
        for j in range(_NJ):
            idx_bj = idx.at[b].at[j]
            for i in range(_G // 16):
                v = iv_b[pl.ds(j * _G + i * 16, 16)]
                bins = jnp.minimum(
                    lax.shift_right_logical(v * 37450, 18), _NUM_BINS - 1
                )
                idx_bj[pl.ds(i * 16, 16)] = bins
        for j in range(_NJ):
            pltpu.async_copy(
                tab_hbm.at[idx.at[b].at[j]],
                rows.at[b].at[pl.ds(j * _G, _G)],
                gsems[b],
            )

    def drain_gathers(b):
        for j in range(_NJ):
            pltpu.make_async_copy(
                tab_hbm.at[idx.at[b].at[j]],
                rows.at[b].at[pl.ds(j * _G, _G)],
                gsems[b],
            ).wait()

    def fire_out(g, b):
        for e in range(_CB):
            pltpu.async_copy(
                rows.at[b].at[pl.ds(e * _HIST, _HIST)],
                out_hbm.at[batch0 + g * _CB + e],
                osems[b],
            )

    def drain_out(g, b):
        for e in range(_CB):
            pltpu.make_async_copy(
                rows.at[b].at[pl.ds(e * _HIST, _HIST)],
                out_hbm.at[batch0 + g * _CB + e],
                osems[b],
            ).wait()

    stage(0, 0)
    stage(1, 1)

    def loop_body(i, carry):
        g = i * 2
        drain_gathers(0)
        fire_out(g, 0)
        drain_gathers(1)
        fire_out(g + 1, 1)

        @pl.when(g + 2 < _NB)
        def _():
            drain_out(g, 0)
            stage(g + 2, 0)
            drain_out(g + 1, 1)
            stage(g + 3, 1)

        return carry

    lax.fori_loop(0, _NB // 2, loop_body, 0)
    drain_out(_NB - 2, 0)
    drain_out(_NB - 1, 1)


_sc_lookup = functools.partial(
    pl.kernel,
    out_type=jax.ShapeDtypeStruct((_BATCH, _HIST, _D), jnp.float32),
    mesh=plsc.VectorSubcoreMesh(core_axis_name="c", subcore_axis_name="s"),
    compiler_params=pltpu.CompilerParams(use_tc_tiling_on_sc=False),
    scratch_types=[
        pltpu.VMEM((2, _R), jnp.int32),
        pltpu.VMEM((2, _NJ, _G), jnp.int32),
        pltpu.VMEM((2, _R, _D), jnp.float32),
        pltpu.SemaphoreType.DMA,
        pltpu.SemaphoreType.DMA,
        pltpu.SemaphoreType.DMA,
        pltpu.SemaphoreType.DMA,
    ],
)(_body)


def _impl(intervals, embed_weight):
    return _sc_lookup(intervals.reshape(_TOTAL), embed_weight)


@functools.lru_cache(maxsize=None)
def _jitted():
    fmt = jax.experimental.layout.Format(
        jax.experimental.layout.Layout(major_to_minor=(0, 1, 2), tiling=()),
        jax.sharding.SingleDeviceSharding(jax.devices()[0]),
    )
    return jax.jit(_impl, out_shardings=fmt)


def kernel(intervals, embed_weight):
    return _jitted()(intervals, embed_weight)


# speedup vs baseline: 1.0507x; 1.0493x over previous
import functools

import jax
import jax.numpy as jnp
from jax import lax
from jax.experimental import pallas as pl
from jax.experimental.pallas import tpu as pltpu
from jax.experimental.pallas import tpu_sc as plsc

_NUM_BINS = 1000
_D = 64
_BATCH = 16384
_HIST = 200
_TOTAL = _BATCH * _HIST

_NC = 2
_NS = 16
_NW = _NC * _NS
_CW = _BATCH // _NW
_G = 128
_NJ = _CW // _G
_NB = _HIST


def _body(iv_hbm, tab_hbm, out_hbm, iv, idx, rows, gsem0, gsem1, osem0, osem1):
    wid = lax.axis_index("s") * _NC + lax.axis_index("c")
    col0 = wid * _CW
    gsems = (gsem0, gsem1)
    osems = (osem0, osem1)

    def stage(h, b):
        start = h * _BATCH + col0
        pltpu.sync_copy(iv_hbm.at[pl.ds(start, _CW)], iv.at[b])
        iv_b = iv.at[b]
        for j in range(_NJ):
            idx_bj = idx.at[b].at[j]
            for i in range(_G // 16):
                v = iv_b[pl.ds(j * _G + i * 16, 16)]
                bins = jnp.minimum(
                    lax.shift_right_logical(v * 37450, 18), _NUM_BINS - 1
                )
                idx_bj[pl.ds(i * 16, 16)] = bins
        for j in range(_NJ):
            pltpu.async_copy(
                tab_hbm.at[idx.at[b].at[j]],
                rows.at[b].at[pl.ds(j * _G, _G)],
                gsems[b],
            )

    def drain_gathers(b):
        for j in range(_NJ):
            pltpu.make_async_copy(
                tab_hbm.at[idx.at[b].at[j]],
                rows.at[b].at[pl.ds(j * _G, _G)],
                gsems[b],
            ).wait()

    def fire_out(h, b):
        pltpu.async_copy(
            rows.at[b],
            out_hbm.at[h].at[pl.ds(col0, _CW)],
            osems[b],
        )

    def drain_out(h, b):
        pltpu.make_async_copy(
            rows.at[b],
            out_hbm.at[h].at[pl.ds(col0, _CW)],
            osems[b],
        ).wait()

    stage(0, 0)
    stage(1, 1)

    def loop_body(i, carry):
        h = i * 2
        drain_gathers(0)
        fire_out(h, 0)
        drain_gathers(1)
        fire_out(h + 1, 1)

        @pl.when(h + 2 < _NB)
        def _():
            drain_out(h, 0)
            stage(h + 2, 0)
            drain_out(h + 1, 1)
            stage(h + 3, 1)

        return carry

    lax.fori_loop(0, _NB // 2, loop_body, 0)
    drain_out(_NB - 2, 0)
    drain_out(_NB - 1, 1)


_sc_lookup = functools.partial(
    pl.kernel,
    out_type=jax.ShapeDtypeStruct((_HIST, _BATCH, _D), jnp.float32),
    mesh=plsc.VectorSubcoreMesh(core_axis_name="c", subcore_axis_name="s"),
    compiler_params=pltpu.CompilerParams(use_tc_tiling_on_sc=False),
    scratch_types=[
        pltpu.VMEM((2, _CW), jnp.int32),
        pltpu.VMEM((2, _NJ, _G), jnp.int32),
        pltpu.VMEM((2, _CW, _D), jnp.float32),
        pltpu.SemaphoreType.DMA,
        pltpu.SemaphoreType.DMA,
        pltpu.SemaphoreType.DMA,
        pltpu.SemaphoreType.DMA,
    ],
)(_body)


def _impl(intervals, embed_weight):
    t_out = _sc_lookup(intervals.T.reshape(_TOTAL), embed_weight)
    return t_out.transpose(1, 0, 2)


kernel = jax.jit(_impl)
